# baseline (speedup 1.0000x reference)
import functools

import jax
import jax.numpy as jnp
from jax import lax
from jax.experimental import pallas as pl
from jax.experimental.pallas import tpu as pltpu
from jax.experimental.pallas import tpu_sc as plsc

BATCH = 4096
HIST = 50
NUM_ITEMS = 10
EMBED_DIM = 128

_INFO = plsc.get_sparse_core_info()
_NC = _INFO.num_cores
_NS = _INFO.num_subcores
_NW = _NC * _NS

_B_PER_W = BATCH // _NW
_CHUNK = _B_PER_W
_K = 5
_NWIN = HIST // _K


def _sc_lookup(x_blocks, table):
    mesh = plsc.VectorSubcoreMesh(core_axis_name="c", subcore_axis_name="s")

    @functools.partial(
        pl.kernel,
        mesh=mesh,
        out_type=jax.ShapeDtypeStruct((HIST, BATCH, EMBED_DIM), jnp.float32),
        scratch_types=[
            pltpu.VMEM_SHARED((_NS * NUM_ITEMS, EMBED_DIM), jnp.float32),
            pltpu.VMEM((HIST, _CHUNK), jnp.int32),
            pltpu.VMEM((_K, _CHUNK, EMBED_DIM), jnp.float32),
        ]
        + [pltpu.SemaphoreType.DMA] * (2 * _K),
    )
    def k(x_hbm, table_hbm, out_hbm, tab_sh, idx_v, bufs_v, *sems):
        gsem = sems[:_K]
        ssem = sems[_K:]
        sid = lax.axis_index("s")
        wid = sid * _NC + lax.axis_index("c")

        pltpu.sync_copy(table_hbm, tab_sh.at[pl.ds(sid * NUM_ITEMS, NUM_ITEMS)])
        pltpu.sync_copy(x_hbm.at[wid], idx_v)

        b0 = wid * _B_PER_W

        def window(g, carry):
            gh = []
            for b in range(_K):
                @pl.when(g > 0)
                def _(b=b):
                    pltpu.make_async_copy(
                        bufs_v.at[b], out_hbm.at[0, pl.ds(0, _CHUNK)], ssem[b]
                    ).wait()

                gh.append(
                    pltpu.async_copy(
                        tab_sh.at[idx_v.at[g * _K + b]], bufs_v.at[b], gsem[b]
                    )
                )
            for b in range(_K):
                gh[b].wait()
                pltpu.async_copy(
                    bufs_v.at[b],
                    out_hbm.at[g * _K + b, pl.ds(b0, _CHUNK)],
                    ssem[b],
                )
            return carry

        lax.fori_loop(0, _NWIN, window, 0, unroll=False)

        for b in range(_K):
            pltpu.make_async_copy(
                bufs_v.at[b], out_hbm.at[0, pl.ds(0, _CHUNK)], ssem[b]
            ).wait()

    return k(x_blocks, table)


def kernel(x, table):
    ids = x.astype(jnp.int32)
    ids = ids.T.reshape(HIST, _NW, _B_PER_W).transpose(1, 0, 2)
    sub = (jnp.arange(_NW, dtype=jnp.int32) // _NC) * NUM_ITEMS
    ids = ids + sub[:, None, None]
    out = _sc_lookup(ids, table)
    return out.transpose(1, 0, 2)
